# baseline (speedup 1.0000x reference)
import functools

import jax
import jax.numpy as jnp
from jax import lax
from jax.experimental import pallas as pl
from jax.experimental.pallas import tpu as pltpu
from jax.experimental.pallas import tpu_sc as plsc

NUM_LEVELS = 16
BASE_RES = 16
MAX_RES = 2048
FEATS = 2
HASH_SIZE = 524288
D0, D1 = 64, 32
N_PTS = 1048576


def _level_params(level):
    resolution = int(BASE_RES * (MAX_RES / BASE_RES) ** (level / (NUM_LEVELS - 1)))
    hash_size = min(resolution ** 3, HASH_SIZE)
    return resolution, hash_size


_LEVELS = [_level_params(l) for l in range(NUM_LEVELS)]


_HASH_BN = 8192
_RW = 8
_RSH = 2
_RMSK = _RW // 2 - 1


def _hash_body(coords_ref, idx_ref):
    c = (coords_ref[...] + 1.0) / 2.0
    for lvl in range(NUM_LEVELS):
        res, hs = _LEVELS[lvl]
        cd = jnp.clip(jnp.floor(c * res).astype(jnp.int32), 0, res - 1)
        x, y, z = cd[0], cd[1], cd[2]
        if hs == res ** 3:
            idx = (x * res) * res + y * res + z
        else:
            m = hs - 1
            idx = (((x * res) & m) * res + y * res + z) & m
        idx_ref[0, lvl, :] = idx >> _RSH
        idx_ref[1, lvl, :] = (idx & _RMSK) * 2


def _hash_indices(coords_t):
    grid = N_PTS // _HASH_BN
    return pl.pallas_call(
        _hash_body,
        grid=(grid,),
        in_specs=[pl.BlockSpec((3, _HASH_BN), lambda i: (0, i))],
        out_specs=pl.BlockSpec((2, NUM_LEVELS, _HASH_BN), lambda i: (0, 0, i)),
        out_shape=jax.ShapeDtypeStruct((2, NUM_LEVELS, N_PTS), jnp.int32),
    )(coords_t)


_NC = 2
_NS = 16
_NW = _NC * _NS
_BPW = N_PTS // _NW
_GCH = 1024
_NBUF = 4


def _gather_body(idx_hbm, *rest):
    tables = rest[:NUM_LEVELS]
    out_hbm = rest[NUM_LEVELS]
    scratch = rest[NUM_LEVELS + 1:]
    tbl_s = scratch[0]
    row_v = scratch[1:1 + _NBUF]
    col_v = scratch[1 + _NBUF:1 + 2 * _NBUF]
    rbuf = scratch[1 + 2 * _NBUF:1 + 3 * _NBUF]
    e_v = scratch[1 + 3 * _NBUF:1 + 4 * _NBUF]
    o_v = scratch[1 + 4 * _NBUF:1 + 5 * _NBUF]
    gsem = scratch[1 + 5 * _NBUF:1 + 6 * _NBUF]
    osem = scratch[1 + 6 * _NBUF:1 + 7 * _NBUF]
    sid = lax.axis_index("s")
    wid = sid * _NC + lax.axis_index("c")
    base = wid * _BPW
    nch = _BPW // _GCH
    for lvl in range(NUM_LEVELS):
        _, hs = _LEVELS[lvl]
        nrows = 2 * hs // _RW
        plsc.subcore_barrier()

        @pl.when(sid == 0)
        def _stage(lvl=lvl, nrows=nrows):
            pltpu.sync_copy(tables[lvl], tbl_s.at[pl.ds(0, nrows)])

        plsc.subcore_barrier()

        @pl.loop(0, nch, step=_NBUF)
        def _group(j0, lvl=lvl):
            for b in range(_NBUF):
                off = base + (j0 + b) * _GCH
                pltpu.sync_copy(idx_hbm.at[0, lvl, pl.ds(off, _GCH)], row_v[b])
                pltpu.sync_copy(idx_hbm.at[1, lvl, pl.ds(off, _GCH)], col_v[b])
                pltpu.async_copy(tbl_s.at[row_v[b]], rbuf[b], gsem[b])
            for b in range(_NBUF):
                off = base + (j0 + b) * _GCH
                pltpu.make_async_copy(tbl_s.at[row_v[b]], rbuf[b],
                                      gsem[b]).wait()

                @pl.loop(0, _GCH // 16)
                def _extract(k, b=b):
                    rows16 = k * 16 + lax.iota(jnp.int32, 16)
                    cols16 = col_v[b][pl.ds(k * 16, 16)]
                    e = plsc.load_gather(rbuf[b], [rows16, cols16])
                    o = plsc.load_gather(rbuf[b], [rows16, cols16 + 1])
                    e_v[b][pl.ds(k * 16, 16)] = e
                    o_v[b][pl.ds(k * 16, 16)] = o

                pltpu.async_copy(e_v[b], out_hbm.at[0, lvl, pl.ds(off, _GCH)],
                                 osem[b])
                pltpu.async_copy(o_v[b], out_hbm.at[1, lvl, pl.ds(off, _GCH)],
                                 osem[b])
            for b in range(_NBUF):
                off = base + (j0 + b) * _GCH
                pltpu.make_async_copy(e_v[b], out_hbm.at[0, lvl, pl.ds(off, _GCH)],
                                      osem[b]).wait()
                pltpu.make_async_copy(o_v[b], out_hbm.at[1, lvl, pl.ds(off, _GCH)],
                                      osem[b]).wait()


def _sc_gather(idx, tables_flat):
    mesh = plsc.VectorSubcoreMesh(core_axis_name="c", subcore_axis_name="s")
    k = functools.partial(
        pl.kernel,
        mesh=mesh,
        compiler_params=pltpu.CompilerParams(use_tc_tiling_on_sc=False,
                                           needs_layout_passes=False),
        out_type=jax.ShapeDtypeStruct((FEATS, NUM_LEVELS, N_PTS), jnp.float32),
        scratch_types=(
            [pltpu.VMEM_SHARED((2 * HASH_SIZE // _RW, _RW), jnp.float32)]
            + [pltpu.VMEM((_GCH,), jnp.int32) for _ in range(2 * _NBUF)]
            + [pltpu.VMEM((_GCH, _RW), jnp.float32) for _ in range(_NBUF)]
            + [pltpu.VMEM((_GCH,), jnp.float32) for _ in range(2 * _NBUF)]
            + [pltpu.SemaphoreType.DMA for _ in range(2 * _NBUF)]
        ),
    )(_gather_body)
    return k(idx, *tables_flat)


_MLP_BN = 2048


def _mlp_body(f_ref, w0_ref, b0_ref, w1_ref, b1_ref, out_ref):
    xt = f_ref[...].reshape(NUM_LEVELS * FEATS, _MLP_BN)
    h = jnp.maximum(
        lax.dot_general(xt, w0_ref[...], (((0,), (0,)), ((), ())),
                        preferred_element_type=jnp.float32) + b0_ref[...],
        0.0,
    )
    out_ref[...] = (
        jnp.dot(h, w1_ref[...], preferred_element_type=jnp.float32) + b1_ref[...]
    )


def _mlp(feats, W0, b0, W1, b1):
    W0p = W0.reshape(NUM_LEVELS, FEATS, D0).transpose(1, 0, 2).reshape(
        NUM_LEVELS * FEATS, D0)
    grid = N_PTS // _MLP_BN
    return pl.pallas_call(
        _mlp_body,
        grid=(grid,),
        in_specs=[
            pl.BlockSpec((FEATS, NUM_LEVELS, _MLP_BN), lambda i: (0, 0, i)),
            pl.BlockSpec((NUM_LEVELS * FEATS, D0), lambda i: (0, 0)),
            pl.BlockSpec((D0,), lambda i: (0,)),
            pl.BlockSpec((D0, D1), lambda i: (0, 0)),
            pl.BlockSpec((D1,), lambda i: (0,)),
        ],
        out_specs=pl.BlockSpec((_MLP_BN, D1), lambda i: (i, 0)),
        out_shape=jax.ShapeDtypeStruct((N_PTS, D1), jnp.float32),
    )(feats, W0p, b0, W1, b1)


def kernel(coords, W0, b0, W1, b1, table_0, table_1, table_2, table_3,
           table_4, table_5, table_6, table_7, table_8, table_9, table_10,
           table_11, table_12, table_13, table_14, table_15):
    tables = [table_0, table_1, table_2, table_3, table_4, table_5, table_6,
              table_7, table_8, table_9, table_10, table_11, table_12,
              table_13, table_14, table_15]
    idx = _hash_indices(coords.T)
    feats = _sc_gather(idx, [t.reshape(-1, _RW) for t in tables])
    return _mlp(feats, W0, b0, W1, b1)


# speedup vs baseline: 1.0862x; 1.0862x over previous
import functools

import jax
import jax.numpy as jnp
from jax import lax
from jax.experimental import pallas as pl
from jax.experimental.pallas import tpu as pltpu
from jax.experimental.pallas import tpu_sc as plsc

NUM_LEVELS = 16
BASE_RES = 16
MAX_RES = 2048
FEATS = 2
HASH_SIZE = 524288
D0, D1 = 64, 32
N_PTS = 1048576


def _level_params(level):
    resolution = int(BASE_RES * (MAX_RES / BASE_RES) ** (level / (NUM_LEVELS - 1)))
    hash_size = min(resolution ** 3, HASH_SIZE)
    return resolution, hash_size


_LEVELS = [_level_params(l) for l in range(NUM_LEVELS)]


_HASH_BN = 8192


def _hash_body(coords_ref, idx_ref):
    c = (coords_ref[...] + 1.0) / 2.0
    for lvl in range(NUM_LEVELS):
        res, hs = _LEVELS[lvl]
        cd = jnp.clip(jnp.floor(c * res).astype(jnp.int32), 0, res - 1)
        x, y, z = cd[0], cd[1], cd[2]
        if hs == res ** 3:
            idx = (x * res) * res + y * res + z
        else:
            m = hs - 1
            idx = (((x * res) & m) * res + y * res + z) & m
        e = idx * 2
        idx_ref[0, lvl, :] = e
        idx_ref[1, lvl, :] = e + 1


def _hash_indices(coords_t):
    grid = N_PTS // _HASH_BN
    return pl.pallas_call(
        _hash_body,
        grid=(grid,),
        in_specs=[pl.BlockSpec((3, _HASH_BN), lambda i: (0, i))],
        out_specs=pl.BlockSpec((2, NUM_LEVELS, _HASH_BN), lambda i: (0, 0, i)),
        out_shape=jax.ShapeDtypeStruct((2, NUM_LEVELS, N_PTS), jnp.int32),
    )(coords_t)


_NC = 2
_NS = 16
_NW = _NC * _NS
_BPW = N_PTS // _NW
_GCH = 2048
_NBUF = 4


def _gather_body(idx_hbm, *rest):
    tables = rest[:NUM_LEVELS]
    out_hbm = rest[NUM_LEVELS]
    scratch = rest[NUM_LEVELS + 1:]
    tbl_s = scratch[0]
    idxe_v = scratch[1:1 + _NBUF]
    idxo_v = scratch[1 + _NBUF:1 + 2 * _NBUF]
    e_v = scratch[1 + 2 * _NBUF:1 + 3 * _NBUF]
    o_v = scratch[1 + 3 * _NBUF:1 + 4 * _NBUF]
    isem = scratch[1 + 4 * _NBUF:1 + 5 * _NBUF]
    gsem = scratch[1 + 5 * _NBUF:1 + 6 * _NBUF]
    osem = scratch[1 + 6 * _NBUF:1 + 7 * _NBUF]
    sid = lax.axis_index("s")
    wid = sid * _NC + lax.axis_index("c")
    base = wid * _BPW
    nch = _BPW // _GCH

    def _fire_idx(lvl, j, b):
        off = base + j * _GCH
        pltpu.async_copy(idx_hbm.at[0, lvl, pl.ds(off, _GCH)], idxe_v[b], isem[b])
        pltpu.async_copy(idx_hbm.at[1, lvl, pl.ds(off, _GCH)], idxo_v[b], isem[b])

    def _wait_idx(lvl, j, b):
        off = base + j * _GCH
        pltpu.make_async_copy(idx_hbm.at[0, lvl, pl.ds(off, _GCH)], idxe_v[b],
                              isem[b]).wait()
        pltpu.make_async_copy(idx_hbm.at[1, lvl, pl.ds(off, _GCH)], idxo_v[b],
                              isem[b]).wait()

    for b in range(_NBUF):
        _fire_idx(0, b, b)

    for lvl in range(NUM_LEVELS):
        _, hs = _LEVELS[lvl]
        nrows = 2 * hs
        plsc.subcore_barrier()

        @pl.when(sid == 0)
        def _stage(lvl=lvl, nrows=nrows):
            pltpu.sync_copy(tables[lvl], tbl_s.at[pl.ds(0, nrows)])

        plsc.subcore_barrier()

        @pl.loop(0, nch, step=_NBUF)
        def _group(j0, lvl=lvl):
            for b in range(_NBUF):
                j = j0 + b
                off = base + j * _GCH
                _wait_idx(lvl, j, b)
                pltpu.async_copy(tbl_s.at[idxe_v[b]], e_v[b], gsem[b])
                pltpu.async_copy(tbl_s.at[idxo_v[b]], o_v[b], gsem[b])
            for b in range(_NBUF):
                j = j0 + b
                off = base + j * _GCH
                pltpu.make_async_copy(tbl_s.at[idxe_v[b]], e_v[b], gsem[b]).wait()
                pltpu.make_async_copy(tbl_s.at[idxo_v[b]], o_v[b], gsem[b]).wait()
                nj = j + _NBUF
                nlvl = lvl + 1 if lvl + 1 < NUM_LEVELS else lvl
                if lvl + 1 < NUM_LEVELS or True:
                    @pl.when(nj < nch)
                    def _pf_same(j=j, b=b, lvl=lvl):
                        _fire_idx(lvl, j + _NBUF, b)
                    if lvl + 1 < NUM_LEVELS:
                        @pl.when(nj >= nch)
                        def _pf_next(j=j, b=b, nlvl=nlvl):
                            _fire_idx(nlvl, j + _NBUF - nch, b)
                pltpu.async_copy(e_v[b], out_hbm.at[0, lvl, pl.ds(off, _GCH)],
                                 osem[b])
                pltpu.async_copy(o_v[b], out_hbm.at[1, lvl, pl.ds(off, _GCH)],
                                 osem[b])
            for b in range(_NBUF):
                off = base + (j0 + b) * _GCH
                pltpu.make_async_copy(e_v[b], out_hbm.at[0, lvl, pl.ds(off, _GCH)],
                                      osem[b]).wait()
                pltpu.make_async_copy(o_v[b], out_hbm.at[1, lvl, pl.ds(off, _GCH)],
                                      osem[b]).wait()


def _sc_gather(idx, tables_flat):
    mesh = plsc.VectorSubcoreMesh(core_axis_name="c", subcore_axis_name="s")
    k = functools.partial(
        pl.kernel,
        mesh=mesh,
        compiler_params=pltpu.CompilerParams(use_tc_tiling_on_sc=False,
                                           needs_layout_passes=False),
        out_type=jax.ShapeDtypeStruct((FEATS, NUM_LEVELS, N_PTS), jnp.float32),
        scratch_types=(
            [pltpu.VMEM_SHARED((2 * HASH_SIZE,), jnp.float32)]
            + [pltpu.VMEM((_GCH,), jnp.int32) for _ in range(2 * _NBUF)]
            + [pltpu.VMEM((_GCH,), jnp.float32) for _ in range(2 * _NBUF)]
            + [pltpu.SemaphoreType.DMA for _ in range(3 * _NBUF)]
        ),
    )(_gather_body)
    return k(idx, *tables_flat)


_MLP_BN = 2048


def _mlp_body(f_ref, w0_ref, b0_ref, w1_ref, b1_ref, out_ref):
    xt = f_ref[...].reshape(NUM_LEVELS * FEATS, _MLP_BN)
    h = jnp.maximum(
        lax.dot_general(xt, w0_ref[...], (((0,), (0,)), ((), ())),
                        preferred_element_type=jnp.float32) + b0_ref[...],
        0.0,
    )
    out_ref[...] = (
        jnp.dot(h, w1_ref[...], preferred_element_type=jnp.float32) + b1_ref[...]
    )


def _mlp(feats, W0, b0, W1, b1):
    W0p = W0.reshape(NUM_LEVELS, FEATS, D0).transpose(1, 0, 2).reshape(
        NUM_LEVELS * FEATS, D0)
    grid = N_PTS // _MLP_BN
    return pl.pallas_call(
        _mlp_body,
        grid=(grid,),
        in_specs=[
            pl.BlockSpec((FEATS, NUM_LEVELS, _MLP_BN), lambda i: (0, 0, i)),
            pl.BlockSpec((NUM_LEVELS * FEATS, D0), lambda i: (0, 0)),
            pl.BlockSpec((D0,), lambda i: (0,)),
            pl.BlockSpec((D0, D1), lambda i: (0, 0)),
            pl.BlockSpec((D1,), lambda i: (0,)),
        ],
        out_specs=pl.BlockSpec((_MLP_BN, D1), lambda i: (i, 0)),
        out_shape=jax.ShapeDtypeStruct((N_PTS, D1), jnp.float32),
    )(feats, W0p, b0, W1, b1)


def kernel(coords, W0, b0, W1, b1, table_0, table_1, table_2, table_3,
           table_4, table_5, table_6, table_7, table_8, table_9, table_10,
           table_11, table_12, table_13, table_14, table_15):
    tables = [table_0, table_1, table_2, table_3, table_4, table_5, table_6,
              table_7, table_8, table_9, table_10, table_11, table_12,
              table_13, table_14, table_15]
    idx = _hash_indices(coords.T)
    feats = _sc_gather(idx, [t.reshape(-1) for t in tables])
    return _mlp(feats, W0, b0, W1, b1)
